# baseline (speedup 1.0000x reference)
import functools

import jax
import jax.numpy as jnp
from jax import lax
from jax.experimental import pallas as pl
from jax.experimental.pallas import tpu as pltpu
from jax.experimental.pallas import tpu_sc as plsc

F32 = jnp.float32
I32 = jnp.int32

NS = 16
LANES = 16
NBR = 4
NBI = 8


@functools.lru_cache(maxsize=None)
def _gcn_sc(n_user, n_nodes, dh, e_pad, batch, n_layer):
    CH = e_pad // (NS * 128)
    RP = n_nodes // NS
    ZC = 25 if RP % 25 == 0 else RP
    BT = batch // NS
    NH = dh // LANES
    L1 = n_layer + 1
    assert n_nodes % NS == 0 and RP % ZC == 0
    assert batch % (NS * 128) == 0 and dh % LANES == 0
    assert CH % NBI == 0 and CH >= NBI

    mesh = plsc.VectorSubcoreMesh(core_axis_name="c", subcore_axis_name="s")

    @functools.partial(
        pl.kernel,
        out_type=(jax.ShapeDtypeStruct((2, L1, batch, dh), F32),
                  jax.ShapeDtypeStruct((2, L1, batch, dh), F32)),
        mesh=mesh,
        scratch_types=[
            pltpu.HBM((2 * n_nodes, dh), F32),
            pltpu.HBM((2 * n_nodes, dh), F32),
            pltpu.VMEM_SHARED((n_nodes, dh), F32),
            pltpu.VMEM((NBI, 2, 128), I32),
            pltpu.VMEM((NBI, 128), F32),
            pltpu.VMEM((NBR, 128, dh), F32),
            pltpu.VMEM((ZC, dh), F32),
            pltpu.VMEM((4, 128), I32),
            pltpu.VMEM((4, 128), I32),
            pltpu.SemaphoreType.DMA((NBI,)),
            pltpu.SemaphoreType.DMA((NBR,)),
            pltpu.SemaphoreType.DMA((NBR,)),
            pltpu.SemaphoreType.DMA,
            pltpu.SemaphoreType.DMA,
        ],
        compiler_params=pltpu.CompilerParams(use_tc_tiling_on_sc=False),
    )
    def k(t0, meta, wp, idxu, idxi, us_o, is_o,
          t1, t2, acc, ibuf, wring, rows, zbuf, bx_t, bx_a,
          lsem, gsem, ssem, zsem, bsem):
        c = lax.axis_index("c")
        s = lax.axis_index("s")
        coff = c * n_nodes
        cov = lax.broadcast(jnp.asarray(coff, I32), (LANES,))
        zv = jnp.zeros((LANES,), F32)

        def zrow(r, _):
            for h in range(NH):
                zbuf[r, pl.ds(h * LANES, LANES)] = zv
            return 0
        lax.fori_loop(0, ZC, zrow, 0)

        def zero_acc():
            r0 = s * RP

            def zgo(q, _):
                pltpu.async_copy(zbuf, acc.at[pl.ds(r0 + q * ZC, ZC)], zsem)
                return 0
            lax.fori_loop(0, RP // ZC, zgo, 0)

            def zwait(q, _):
                pltpu.make_async_copy(zbuf, acc.at[pl.ds(r0, ZC)], zsem).wait()
                return 0
            lax.fori_loop(0, RP // ZC, zwait, 0)

        def add_off(idxref, off_v):
            for g in range(128 // LANES):
                idxref[pl.ds(g * LANES, LANES)] = (
                    idxref[pl.ds(g * LANES, LANES)] + off_v)

        BSPEC = ((0, 0), (0, 1), (1, 0), (1, 1))

        def load_batch_idx():
            b0 = s * BT
            for q, (side, half) in enumerate(BSPEC):
                src_idx = idxi if side else idxu
                pltpu.sync_copy(src_idx.at[pl.ds(b0 + half * 128, 128)],
                                bx_a.at[q])
                if side:
                    add_off(bx_a.at[q], jnp.full((LANES,), n_user, I32))
                for g in range(128 // LANES):
                    d = pl.ds(g * LANES, LANES)
                    bx_t[q, d] = bx_a[q, d] + cov

        def batch_out(tbl, bx, li):
            b0 = s * BT
            for q in range(4):
                pltpu.async_copy(tbl.at[bx.at[q]], rows.at[q % NBR],
                                 gsem.at[q % NBR])
            for q, (side, half) in enumerate(BSPEC):
                out = is_o if side else us_o
                pltpu.make_async_copy(tbl.at[bx.at[q]], rows.at[q % NBR],
                                      gsem.at[q % NBR]).wait()
                pltpu.async_copy(rows.at[q % NBR],
                                 out.at[c, li, pl.ds(b0 + half * 128, 128)],
                                 bsem)
            for q in range(4):
                pltpu.make_async_copy(
                    rows.at[0], us_o.at[c, li, pl.ds(b0, 128)], bsem).wait()

        def edge_pass(src):
            cbase = s * CH

            def start_loads(j, bi):
                pltpu.async_copy(meta.at[cbase + j], ibuf.at[bi],
                                 lsem.at[bi])
                pltpu.async_copy(wp.at[pl.ds((cbase + j) * 128, 128)],
                                 wring.at[bi], lsem.at[bi])

            def prep_gather(bi, br):
                pltpu.make_async_copy(meta.at[cbase], ibuf.at[bi],
                                      lsem.at[bi]).wait()
                pltpu.make_async_copy(wp.at[pl.ds(0, 128)],
                                      wring.at[bi], lsem.at[bi]).wait()
                add_off(ibuf.at[bi, 0], cov)
                pltpu.async_copy(src.at[ibuf.at[bi, 0]], rows.at[br],
                                 gsem.at[br])

            def wait_gather(bi, br):
                pltpu.make_async_copy(src.at[ibuf.at[bi, 0]], rows.at[br],
                                      gsem.at[br]).wait()

            def start_scatter(bi, br):
                pltpu.async_copy(rows.at[br], acc.at[ibuf.at[bi, 1]],
                                 ssem.at[br], add=True)

            def drain_scatter(bi, br):
                pltpu.make_async_copy(rows.at[br], acc.at[ibuf.at[bi, 1]],
                                      ssem.at[br]).wait()

            def multiply(bi, br):
                def medge(g, _):
                    wv = wring[bi, pl.ds(g * LANES, LANES)]
                    for e in range(LANES):
                        w = lax.broadcast(wv[e], (LANES,))
                        r = g * LANES + e
                        for h in range(NH):
                            d = pl.ds(h * LANES, LANES)
                            rows[br, r, d] = rows[br, r, d] * w
                    return 0
                lax.fori_loop(0, 128 // LANES, medge, 0)

            for j in range(5):
                start_loads(j, j)
            for j in range(3):
                prep_gather(j, j % NBR)

            def group(g, _):
                for t in range(NBI):
                    j = g * NBI + t
                    br = t % NBR

                    @pl.when(j >= 1)
                    def _():
                        drain_scatter((t - 1) % NBI, (t - 1) % NBR)

                    @pl.when(j + 3 < CH)
                    def _():
                        prep_gather((t + 3) % NBI, (t + 3) % NBR)

                    @pl.when(j + 5 < CH)
                    def _():
                        start_loads(j + 5, (t + 5) % NBI)

                    wait_gather(t, br)
                    multiply(t, br)
                    start_scatter(t, br)
                return 0
            lax.fori_loop(0, CH // NBI, group, 0)
            drain_scatter((CH - 1) % NBI, (CH - 1) % NBR)

        def dump_start(dst):
            r0 = s * RP
            pltpu.async_copy(acc.at[pl.ds(r0, RP)],
                             dst.at[pl.ds(coff + r0, RP)], zsem)

        def dump_wait(dst):
            r0 = s * RP
            pltpu.make_async_copy(acc.at[pl.ds(r0, RP)],
                                  dst.at[pl.ds(coff + r0, RP)], zsem).wait()

        load_batch_idx()
        zero_acc()
        batch_out(t0, bx_t, 0)
        plsc.subcore_barrier()
        tables = [(t0, t1), (t1, t2), (t2, t1)]
        for li in range(n_layer):
            src, dst = tables[li % 3]
            edge_pass(src)
            plsc.subcore_barrier()
            if li + 1 < n_layer:
                dump_start(dst)
                batch_out(acc, bx_a, li + 1)
                dump_wait(dst)
                plsc.subcore_barrier()
                zero_acc()
                plsc.subcore_barrier()
            else:
                batch_out(acc, bx_a, li + 1)

    return k


def _dot_body(scale, l1, u_ref, i_ref, o_ref):
    acc = None
    for c in range(2):
        us = u_ref[c, 0]
        is_ = i_ref[c, 0]
        for l in range(1, l1):
            us = us + u_ref[c, l]
            is_ = is_ + i_ref[c, l]
        p = us * is_
        acc = p if c == 0 else acc + p
    o_ref[...] = jnp.sum(acc, axis=1) * scale


def kernel(embeds_u, embeds_i, edge_weight, edge_index, idx_u, idx_i):
    n_user, d = embeds_u.shape
    n_nodes = n_user + embeds_i.shape[0]
    e = edge_index.shape[1]
    batch = idx_u.shape[0]
    dh = d // 2
    n_layer = 3

    all_emb = jnp.concatenate([embeds_u, embeds_i], axis=0).astype(F32)
    t0 = jnp.concatenate([all_emb[:, :dh], all_emb[:, dh:]], axis=0)

    ch = -(-e // (NS * 128))
    ch += (-ch) % NBI
    e_pad = ch * NS * 128
    pad = e_pad - e
    row = edge_index[0].astype(I32)
    col = edge_index[1].astype(I32)
    w = edge_weight.astype(F32)
    if pad:
        row = jnp.concatenate([row, jnp.zeros((pad,), I32)])
        col = jnp.concatenate([col, jnp.zeros((pad,), I32)])
        w = jnp.concatenate([w, jnp.zeros((pad,), F32)])
    meta = jnp.stack([col.reshape(-1, 128), row.reshape(-1, 128)],
                     axis=1)

    us, is_ = _gcn_sc(n_user, n_nodes, dh, e_pad, batch, n_layer)(
        t0, meta, w, idx_u.astype(I32), idx_i.astype(I32))

    scale = 1.0 / float((n_layer + 1) ** 2)
    return pl.pallas_call(
        functools.partial(_dot_body, scale, n_layer + 1),
        out_shape=jax.ShapeDtypeStruct((batch,), F32),
    )(us, is_)


# speedup vs baseline: 1.1183x; 1.1183x over previous
import functools

import jax
import jax.numpy as jnp
from jax import lax
from jax.experimental import pallas as pl
from jax.experimental.pallas import tpu as pltpu
from jax.experimental.pallas import tpu_sc as plsc

F32 = jnp.float32
I32 = jnp.int32

NS = 16
LANES = 16
NBR = 4
NBI = 8


@functools.lru_cache(maxsize=None)
def _gcn_sc(n_user, n_nodes, dh, e_pad, batch, n_layer):
    CH = e_pad // (NS * 128)
    RP = n_nodes // NS
    ZC = 25 if RP % 25 == 0 else RP
    BT = batch // NS
    NH = dh // LANES
    L1 = n_layer + 1
    assert n_nodes % NS == 0 and RP % ZC == 0
    assert batch % (NS * 128) == 0 and dh % LANES == 0
    assert CH % NBI == 0 and CH >= NBI

    mesh = plsc.VectorSubcoreMesh(core_axis_name="c", subcore_axis_name="s")

    @functools.partial(
        pl.kernel,
        out_type=(jax.ShapeDtypeStruct((2, L1, batch, dh), F32),
                  jax.ShapeDtypeStruct((2, L1, batch, dh), F32)),
        mesh=mesh,
        scratch_types=[
            pltpu.HBM((2 * n_nodes, dh), F32),
            pltpu.HBM((2 * n_nodes, dh), F32),
            pltpu.VMEM_SHARED((n_nodes, dh), F32),
            pltpu.VMEM((NBI, 2, 128), I32),
            pltpu.VMEM((NBI, 128), F32),
            pltpu.VMEM((NBR, 128, dh), F32),
            pltpu.VMEM((ZC, dh), F32),
            pltpu.VMEM((4, 128), I32),
            pltpu.VMEM((4, 128), I32),
            pltpu.SemaphoreType.DMA((NBI,)),
            pltpu.SemaphoreType.DMA((NBR,)),
            pltpu.SemaphoreType.DMA((NBR,)),
            pltpu.SemaphoreType.DMA,
            pltpu.SemaphoreType.DMA,
        ],
        compiler_params=pltpu.CompilerParams(use_tc_tiling_on_sc=False),
    )
    def k(t0, meta, wp, idxu, idxi, us_o, is_o,
          t1, t2, acc, ibuf, wring, rows, zbuf, bx_t, bx_a,
          lsem, gsem, ssem, zsem, bsem):
        c = lax.axis_index("c")
        s = lax.axis_index("s")
        coff = c * n_nodes
        cov = lax.broadcast(jnp.asarray(coff, I32), (LANES,))
        zv = jnp.zeros((LANES,), F32)

        def zrow(r, _):
            for h in range(NH):
                zbuf[r, pl.ds(h * LANES, LANES)] = zv
            return 0
        lax.fori_loop(0, ZC, zrow, 0)

        def zero_acc():
            r0 = s * RP

            def zgo(q, _):
                pltpu.async_copy(zbuf, acc.at[pl.ds(r0 + q * ZC, ZC)], zsem)
                return 0
            lax.fori_loop(0, RP // ZC, zgo, 0)

            def zwait(q, _):
                pltpu.make_async_copy(zbuf, acc.at[pl.ds(r0, ZC)], zsem).wait()
                return 0
            lax.fori_loop(0, RP // ZC, zwait, 0)

        def add_off(idxref, off_v):
            for g in range(128 // LANES):
                idxref[pl.ds(g * LANES, LANES)] = (
                    idxref[pl.ds(g * LANES, LANES)] + off_v)

        BSPEC = ((0, 0), (0, 1), (1, 0), (1, 1))

        def load_batch_idx():
            b0 = s * BT
            for q, (side, half) in enumerate(BSPEC):
                src_idx = idxi if side else idxu
                pltpu.sync_copy(src_idx.at[pl.ds(b0 + half * 128, 128)],
                                bx_a.at[q])
                if side:
                    add_off(bx_a.at[q], jnp.full((LANES,), n_user, I32))
                for g in range(128 // LANES):
                    d = pl.ds(g * LANES, LANES)
                    bx_t[q, d] = bx_a[q, d] + cov

        def batch_out(tbl, bx, li):
            b0 = s * BT
            for q in range(4):
                pltpu.async_copy(tbl.at[bx.at[q]], rows.at[q % NBR],
                                 gsem.at[q % NBR])
            for q, (side, half) in enumerate(BSPEC):
                out = is_o if side else us_o
                pltpu.make_async_copy(tbl.at[bx.at[q]], rows.at[q % NBR],
                                      gsem.at[q % NBR]).wait()
                pltpu.async_copy(rows.at[q % NBR],
                                 out.at[c, li, pl.ds(b0 + half * 128, 128)],
                                 bsem)
            for q in range(4):
                pltpu.make_async_copy(
                    rows.at[0], us_o.at[c, li, pl.ds(b0, 128)], bsem).wait()

        def edge_pass(src):
            cbase = s * CH

            def start_loads(j, bi):
                pltpu.async_copy(meta.at[cbase + j], ibuf.at[bi],
                                 lsem.at[bi])
                pltpu.async_copy(wp.at[pl.ds((cbase + j) * 128, 128)],
                                 wring.at[bi], lsem.at[bi])

            def prep_gather(bi, br):
                pltpu.make_async_copy(meta.at[cbase], ibuf.at[bi],
                                      lsem.at[bi]).wait()
                pltpu.make_async_copy(wp.at[pl.ds(0, 128)],
                                      wring.at[bi], lsem.at[bi]).wait()
                add_off(ibuf.at[bi, 0], cov)
                pltpu.async_copy(src.at[ibuf.at[bi, 0]], rows.at[br],
                                 gsem.at[br])

            def wait_gather(bi, br):
                pltpu.make_async_copy(src.at[ibuf.at[bi, 0]], rows.at[br],
                                      gsem.at[br]).wait()

            def start_scatter(bi, br):
                pltpu.async_copy(rows.at[br], acc.at[ibuf.at[bi, 1]],
                                 ssem.at[br], add=True)

            def drain_scatter(bi, br):
                pltpu.make_async_copy(rows.at[br], acc.at[ibuf.at[bi, 1]],
                                      ssem.at[br]).wait()

            def multiply(bi, br):
                def medge(g, _):
                    wv = wring[bi, pl.ds(g * LANES, LANES)]
                    for e in range(LANES):
                        w = lax.broadcast(wv[e], (LANES,))
                        r = g * LANES + e
                        for h in range(NH):
                            d = pl.ds(h * LANES, LANES)
                            rows[br, r, d] = rows[br, r, d] * w
                    return 0
                lax.fori_loop(0, 128 // LANES, medge, 0)

            for j in range(4):
                start_loads(j, j)
            for j in range(2):
                prep_gather(j, j)

            def group(g, _):
                for t in range(NBI):
                    j = g * NBI + t
                    br = t % NBR

                    @pl.when(j >= 2)
                    def _():
                        drain_scatter((t - 2) % NBI, (t - 2) % NBR)

                    @pl.when(j + 2 < CH)
                    def _():
                        prep_gather((t + 2) % NBI, (t + 2) % NBR)

                    @pl.when(j + 4 < CH)
                    def _():
                        start_loads(j + 4, (t + 4) % NBI)

                    wait_gather(t, br)
                    multiply(t, br)
                    start_scatter(t, br)
                return 0
            lax.fori_loop(0, CH // NBI, group, 0)
            drain_scatter((CH - 2) % NBI, (CH - 2) % NBR)
            drain_scatter((CH - 1) % NBI, (CH - 1) % NBR)

        def dump_start(dst):
            r0 = s * RP
            pltpu.async_copy(acc.at[pl.ds(r0, RP)],
                             dst.at[pl.ds(coff + r0, RP)], zsem)

        def dump_wait(dst):
            r0 = s * RP
            pltpu.make_async_copy(acc.at[pl.ds(r0, RP)],
                                  dst.at[pl.ds(coff + r0, RP)], zsem).wait()

        load_batch_idx()
        zero_acc()
        batch_out(t0, bx_t, 0)
        plsc.subcore_barrier()
        tables = [(t0, t1), (t1, t2), (t2, t1)]
        for li in range(n_layer):
            src, dst = tables[li % 3]
            edge_pass(src)
            plsc.subcore_barrier()
            if li + 1 < n_layer:
                dump_start(dst)
                batch_out(acc, bx_a, li + 1)
                dump_wait(dst)
                plsc.subcore_barrier()
                zero_acc()
                plsc.subcore_barrier()
            else:
                batch_out(acc, bx_a, li + 1)

    return k


def _dot_body(scale, l1, u_ref, i_ref, o_ref):
    acc = None
    for c in range(2):
        us = u_ref[c, 0]
        is_ = i_ref[c, 0]
        for l in range(1, l1):
            us = us + u_ref[c, l]
            is_ = is_ + i_ref[c, l]
        p = us * is_
        acc = p if c == 0 else acc + p
    o_ref[...] = jnp.sum(acc, axis=1) * scale


def kernel(embeds_u, embeds_i, edge_weight, edge_index, idx_u, idx_i):
    n_user, d = embeds_u.shape
    n_nodes = n_user + embeds_i.shape[0]
    e = edge_index.shape[1]
    batch = idx_u.shape[0]
    dh = d // 2
    n_layer = 3

    all_emb = jnp.concatenate([embeds_u, embeds_i], axis=0).astype(F32)
    t0 = jnp.concatenate([all_emb[:, :dh], all_emb[:, dh:]], axis=0)

    ch = -(-e // (NS * 128))
    ch += (-ch) % NBI
    e_pad = ch * NS * 128
    pad = e_pad - e
    row = edge_index[0].astype(I32)
    col = edge_index[1].astype(I32)
    w = edge_weight.astype(F32)
    if pad:
        row = jnp.concatenate([row, jnp.zeros((pad,), I32)])
        col = jnp.concatenate([col, jnp.zeros((pad,), I32)])
        w = jnp.concatenate([w, jnp.zeros((pad,), F32)])
    meta = jnp.stack([col.reshape(-1, 128), row.reshape(-1, 128)],
                     axis=1)

    us, is_ = _gcn_sc(n_user, n_nodes, dh, e_pad, batch, n_layer)(
        t0, meta, w, idx_u.astype(I32), idx_i.astype(I32))

    scale = 1.0 / float((n_layer + 1) ** 2)
    return pl.pallas_call(
        functools.partial(_dot_body, scale, n_layer + 1),
        out_shape=jax.ShapeDtypeStruct((batch,), F32),
    )(us, is_)
